# baseline (speedup 1.0000x reference)
import functools

import jax
import jax.numpy as jnp
from jax import lax
from jax.experimental import pallas as pl
from jax.experimental.pallas import tpu as pltpu
from jax.experimental.pallas import tpu_sc as plsc

NC = 2
NS = 16
NW = NC * NS

RB = 2


def _embed_body(idx_hbm, table_hbm, out_hbm, idx_v, rows_v, sem0, sem1):
    bsz, seq = idx_hbm.shape
    d = table_hbm.shape[1]
    wid = lax.axis_index("s") * NC + lax.axis_index("c")
    rows_per_w = bsz // NW
    iters = rows_per_w // RB
    b_lo = wid * rows_per_w
    sems = (sem0, sem1)

    def fire(g, b):
        pltpu.sync_copy(idx_hbm.at[pl.ds(b_lo + g * RB, RB)], idx_v.at[b])
        for i in range(RB):
            pltpu.async_copy(
                table_hbm.at[idx_v.at[b].at[i]],
                rows_v.at[b].at[i],
                sems[b],
            )

    def drain(b):
        for i in range(RB):
            pltpu.make_async_copy(
                table_hbm.at[idx_v.at[b].at[i]],
                rows_v.at[b].at[i],
                sems[b],
            ).wait()

    fire(0, 0)

    def step(g2, carry):
        for b in range(2):
            g = g2 * 2 + b
            nxt = 1 - b

            @pl.when(g + 1 < iters)
            def _():
                fire(g + 1, nxt)

            drain(b)
            pltpu.sync_copy(
                rows_v.at[b],
                out_hbm.at[pl.ds(b_lo + g * RB, RB)],
            )
        return carry

    lax.fori_loop(0, iters // 2, step, 0)


def _embed_call(token_ids, weight):
    bsz, seq = token_ids.shape
    d = weight.shape[1]
    mesh = plsc.VectorSubcoreMesh(
        core_axis_name="c", subcore_axis_name="s", num_cores=NC, num_subcores=NS
    )
    return pl.kernel(
        _embed_body,
        out_type=jax.ShapeDtypeStruct((bsz, seq, d), jnp.float32),
        mesh=mesh,
        scratch_types=[
            pltpu.VMEM((2, RB, seq), jnp.int32),
            pltpu.VMEM((2, RB, seq, d), jnp.float32),
            pltpu.SemaphoreType.DMA,
            pltpu.SemaphoreType.DMA,
        ],
        compiler_params=pltpu.CompilerParams(use_tc_tiling_on_sc=False),
    )(token_ids, weight)


def kernel(token_ids, weight):
    return _embed_call(token_ids.astype(jnp.int32), weight)
